# Initial kernel scaffold; baseline (speedup 1.0000x reference)
import jax
import jax.numpy as jnp
from jax.experimental import pallas as pl


def kernel(t_local, y, L, W, b):
    raise NotImplementedError("write your pallas kernel here")


# speedup vs baseline: 1.3166x; 1.3166x over previous
import functools

import jax
import jax.numpy as jnp
from jax.experimental import pallas as pl

_G = 8


def _body(y_ref, l_ref, bw0_ref, bw1_ref, bw2_ref, bias_ref, out_ref):
    g = y_ref.shape[0]
    x = y_ref[...]
    xt = jnp.concatenate([x[i] for i in range(g)], axis=1)
    lmat = l_ref[...]
    x0 = xt
    x1 = jnp.dot(lmat, x0.astype(jnp.bfloat16),
                 preferred_element_type=jnp.float32)
    x2 = 2.0 * jnp.dot(lmat, x1.astype(jnp.bfloat16),
                       preferred_element_type=jnp.float32) - x0
    acc = jnp.dot(x0.astype(jnp.bfloat16), bw0_ref[...],
                  preferred_element_type=jnp.float32)
    acc += jnp.dot(x1.astype(jnp.bfloat16), bw1_ref[...],
                   preferred_element_type=jnp.float32)
    acc += jnp.dot(x2.astype(jnp.bfloat16), bw2_ref[...],
                   preferred_element_type=jnp.float32)
    out_t = -0.1 * (acc + bias_ref[...])
    out_ref[...] = jnp.stack(
        [out_t[:, i * 32:(i + 1) * 32] for i in range(g)], axis=0)


@functools.partial(jax.jit, static_argnums=0)
def _run(g, y3, lmat, bw0, bw1, bw2, bias2d):
    b, n, d = y3.shape
    return pl.pallas_call(
        _body,
        grid=(b // g,),
        in_specs=[
            pl.BlockSpec((g, n, d), lambda i: (i, 0, 0)),
            pl.BlockSpec((n, n), lambda i: (0, 0)),
            pl.BlockSpec((g * d, g * d), lambda i: (0, 0)),
            pl.BlockSpec((g * d, g * d), lambda i: (0, 0)),
            pl.BlockSpec((g * d, g * d), lambda i: (0, 0)),
            pl.BlockSpec((1, g * d), lambda i: (0, 0)),
        ],
        out_specs=pl.BlockSpec((g, n, d), lambda i: (i, 0, 0)),
        out_shape=jax.ShapeDtypeStruct((b, n, d), jnp.float32),
    )(y3, lmat, bw0, bw1, bw2, bias2d)


def kernel(t_local, y, L, W, b):
    del t_local
    bsz = y.shape[0]
    n = L.shape[0]
    d = W.shape[1]
    m = W.shape[0] // d
    y3 = y.reshape(bsz, n, d)
    eye = jnp.eye(_G, dtype=jnp.float32)
    bws = [jnp.kron(eye, W[k::m, :]).astype(jnp.bfloat16) for k in range(m)]
    bias2d = jnp.tile(b, _G).reshape(1, _G * d)
    out3 = _run(_G, y3, L.astype(jnp.bfloat16), bws[0], bws[1], bws[2],
                bias2d)
    return out3.reshape(bsz, n * d)


# speedup vs baseline: 1.7268x; 1.3116x over previous
import functools

import jax
import jax.numpy as jnp
from jax.experimental import pallas as pl

_G = 32
_D = 32
_C = 256


def _body(y_ref, l_ref, bw0_ref, bw1_ref, bw2_ref, bias_ref, out_ref):
    g = y_ref.shape[0]
    x = y_ref[...]
    xt = jnp.concatenate([x[i] for i in range(g)], axis=1)
    lmat = l_ref[...]
    x0 = xt
    x0b = x0.astype(jnp.bfloat16)
    x1 = jnp.dot(lmat, x0b, preferred_element_type=jnp.float32)
    x1b = x1.astype(jnp.bfloat16)
    x2 = 2.0 * jnp.dot(lmat, x1b, preferred_element_type=jnp.float32) - x0
    x2b = x2.astype(jnp.bfloat16)
    bias = bias_ref[...]
    outs = []
    for j in range(g * _D // _C):
        sl = slice(j * _C, (j + 1) * _C)
        acc = jnp.dot(x0b[:, sl], bw0_ref[...],
                      preferred_element_type=jnp.float32)
        acc += jnp.dot(x1b[:, sl], bw1_ref[...],
                       preferred_element_type=jnp.float32)
        acc += jnp.dot(x2b[:, sl], bw2_ref[...],
                       preferred_element_type=jnp.float32)
        outs.append(-0.1 * (acc + bias))
    out_t = jnp.concatenate(outs, axis=1)
    out_ref[...] = jnp.stack(
        [out_t[:, i * _D:(i + 1) * _D] for i in range(g)], axis=0)


@functools.partial(jax.jit, static_argnums=0)
def _run(g, y3, lmat, bw0, bw1, bw2, bias2d):
    b, n, d = y3.shape
    return pl.pallas_call(
        _body,
        grid=(b // g,),
        in_specs=[
            pl.BlockSpec((g, n, d), lambda i: (i, 0, 0)),
            pl.BlockSpec((n, n), lambda i: (0, 0)),
            pl.BlockSpec((_C, _C), lambda i: (0, 0)),
            pl.BlockSpec((_C, _C), lambda i: (0, 0)),
            pl.BlockSpec((_C, _C), lambda i: (0, 0)),
            pl.BlockSpec((1, _C), lambda i: (0, 0)),
        ],
        out_specs=pl.BlockSpec((g, n, d), lambda i: (i, 0, 0)),
        out_shape=jax.ShapeDtypeStruct((b, n, d), jnp.float32),
    )(y3, lmat, bw0, bw1, bw2, bias2d)


def kernel(t_local, y, L, W, b):
    del t_local
    bsz = y.shape[0]
    n = L.shape[0]
    d = W.shape[1]
    m = W.shape[0] // d
    y3 = y.reshape(bsz, n, d)
    eye = jnp.eye(_C // d, dtype=jnp.float32)
    bws = [jnp.kron(eye, W[k::m, :]).astype(jnp.bfloat16) for k in range(m)]
    bias2d = jnp.tile(b, _C // d).reshape(1, _C)
    out3 = _run(_G, y3, L.astype(jnp.bfloat16), bws[0], bws[1], bws[2],
                bias2d)
    return out3.reshape(bsz, n * d)


# speedup vs baseline: 2.3945x; 1.3866x over previous
import functools

import jax
import jax.numpy as jnp
from jax.experimental import pallas as pl

_G = 32
_D = 32
_C = 256


def _body(y_ref, l_ref, bw0_ref, bw1_ref, bw2_ref, bias_ref, out_ref):
    g = y_ref.shape[0]
    n = l_ref.shape[0]
    x = y_ref[...].reshape(g, n, _D)
    xt = jnp.concatenate([x[i] for i in range(g)], axis=1)
    lmat = l_ref[...]
    x0 = xt
    x0b = x0.astype(jnp.bfloat16)
    x1 = jnp.dot(lmat, x0b, preferred_element_type=jnp.float32)
    x1b = x1.astype(jnp.bfloat16)
    x2 = 2.0 * jnp.dot(lmat, x1b, preferred_element_type=jnp.float32) - x0
    x2b = x2.astype(jnp.bfloat16)
    bias = bias_ref[...]
    outs = []
    for j in range(g * _D // _C):
        sl = slice(j * _C, (j + 1) * _C)
        acc = jnp.dot(x0b[:, sl], bw0_ref[...],
                      preferred_element_type=jnp.float32)
        acc += jnp.dot(x1b[:, sl], bw1_ref[...],
                       preferred_element_type=jnp.float32)
        acc += jnp.dot(x2b[:, sl], bw2_ref[...],
                       preferred_element_type=jnp.float32)
        outs.append(-0.1 * (acc + bias))
    out_t = jnp.concatenate(outs, axis=1)
    out_ref[...] = jnp.stack(
        [out_t[:, i * _D:(i + 1) * _D] for i in range(g)],
        axis=0).reshape(g, n * _D)


@functools.partial(jax.jit, static_argnums=0)
def _run(g, y, lmat, bw0, bw1, bw2, bias2d):
    b, f = y.shape
    return pl.pallas_call(
        _body,
        grid=(b // g,),
        in_specs=[
            pl.BlockSpec((g, f), lambda i: (i, 0)),
            pl.BlockSpec(lmat.shape, lambda i: (0, 0)),
            pl.BlockSpec((_C, _C), lambda i: (0, 0)),
            pl.BlockSpec((_C, _C), lambda i: (0, 0)),
            pl.BlockSpec((_C, _C), lambda i: (0, 0)),
            pl.BlockSpec((1, _C), lambda i: (0, 0)),
        ],
        out_specs=pl.BlockSpec((g, f), lambda i: (i, 0)),
        out_shape=jax.ShapeDtypeStruct((b, f), jnp.float32),
    )(y, lmat, bw0, bw1, bw2, bias2d)


def kernel(t_local, y, L, W, b):
    del t_local
    d = W.shape[1]
    m = W.shape[0] // d
    eye = jnp.eye(_C // d, dtype=jnp.float32)
    bws = [jnp.kron(eye, W[k::m, :]).astype(jnp.bfloat16) for k in range(m)]
    bias2d = jnp.tile(b, _C // d).reshape(1, _C)
    return _run(_G, y, L.astype(jnp.bfloat16), bws[0], bws[1], bws[2],
                bias2d)


# speedup vs baseline: 2.8848x; 1.2048x over previous
import functools

import jax
import jax.numpy as jnp
from jax.experimental import pallas as pl

_G = 64
_D = 32
_C = 256


def _body(y_ref, l_ref, l2_ref, bwa_ref, bw1_ref, bw2_ref, bias_ref,
          out_ref):
    g = y_ref.shape[0]
    n = l_ref.shape[0]
    xb = y_ref[...].astype(jnp.bfloat16).reshape(g, n, _D)
    x0b = jnp.concatenate([xb[i] for i in range(g)], axis=1)
    x1b = jnp.dot(l_ref[...], x0b,
                  preferred_element_type=jnp.float32).astype(jnp.bfloat16)
    zb = jnp.dot(l2_ref[...], x1b,
                 preferred_element_type=jnp.float32).astype(jnp.bfloat16)
    bias = bias_ref[...]
    outs = []
    for j in range(g * _D // _C):
        sl = slice(j * _C, (j + 1) * _C)
        acc = jnp.dot(x0b[:, sl], bwa_ref[...],
                      preferred_element_type=jnp.float32)
        acc += jnp.dot(x1b[:, sl], bw1_ref[...],
                       preferred_element_type=jnp.float32)
        acc += jnp.dot(zb[:, sl], bw2_ref[...],
                       preferred_element_type=jnp.float32)
        outs.append(acc + bias)
    out_t = jnp.concatenate(outs, axis=1)
    out_ref[...] = jnp.stack(
        [out_t[:, i * _D:(i + 1) * _D] for i in range(g)],
        axis=0).reshape(g, n * _D)


@functools.partial(jax.jit, static_argnums=0)
def _run(g, y, lmat, l2mat, bwa, bw1, bw2, bias2d):
    b, f = y.shape
    return pl.pallas_call(
        _body,
        grid=(b // g,),
        in_specs=[
            pl.BlockSpec((g, f), lambda i: (i, 0)),
            pl.BlockSpec(lmat.shape, lambda i: (0, 0)),
            pl.BlockSpec(l2mat.shape, lambda i: (0, 0)),
            pl.BlockSpec((_C, _C), lambda i: (0, 0)),
            pl.BlockSpec((_C, _C), lambda i: (0, 0)),
            pl.BlockSpec((_C, _C), lambda i: (0, 0)),
            pl.BlockSpec((1, _C), lambda i: (0, 0)),
        ],
        out_specs=pl.BlockSpec((g, f), lambda i: (i, 0)),
        out_shape=jax.ShapeDtypeStruct((b, f), jnp.float32),
    )(y, lmat, l2mat, bwa, bw1, bw2, bias2d)


def kernel(t_local, y, L, W, b):
    del t_local
    d = W.shape[1]
    m = W.shape[0] // d
    eye = jnp.eye(_C // d, dtype=jnp.float32)
    w0, w1, w2 = (-0.1 * W[k::m, :] for k in range(m))
    bwa = jnp.kron(eye, w0 - w2).astype(jnp.bfloat16)
    bw1 = jnp.kron(eye, w1).astype(jnp.bfloat16)
    bw2 = jnp.kron(eye, w2).astype(jnp.bfloat16)
    bias2d = jnp.tile(-0.1 * b, _C // d).reshape(1, _C)
    return _run(_G, y, L.astype(jnp.bfloat16),
                (2.0 * L).astype(jnp.bfloat16), bwa, bw1, bw2, bias2d)


# speedup vs baseline: 3.4475x; 1.1951x over previous
import functools

import jax
import jax.numpy as jnp
from jax.experimental import pallas as pl

_G = 128
_D = 32
_C = 256


def _body(y_ref, l_ref, l2_ref, bwa_ref, bw1_ref, bw2_ref, bias_ref,
          out_ref):
    g = y_ref.shape[0]
    n = l_ref.shape[0]
    xb = y_ref[...].astype(jnp.bfloat16).reshape(g, n, _D)
    x0b = jnp.concatenate([xb[i] for i in range(g)], axis=1)
    x1b = jnp.dot(l_ref[...], x0b,
                  preferred_element_type=jnp.float32).astype(jnp.bfloat16)
    zb = jnp.dot(l2_ref[...], x1b,
                 preferred_element_type=jnp.float32).astype(jnp.bfloat16)
    bias = bias_ref[...]
    outs = []
    for j in range(g * _D // _C):
        sl = slice(j * _C, (j + 1) * _C)
        acc = jnp.dot(x0b[:, sl], bwa_ref[...],
                      preferred_element_type=jnp.float32)
        acc += jnp.dot(x1b[:, sl], bw1_ref[...],
                       preferred_element_type=jnp.float32)
        acc += jnp.dot(zb[:, sl], bw2_ref[...],
                       preferred_element_type=jnp.float32)
        outs.append((acc + bias).astype(jnp.bfloat16))
    out_t = jnp.concatenate(outs, axis=1)
    out_ref[...] = jnp.stack(
        [out_t[:, i * _D:(i + 1) * _D] for i in range(g)],
        axis=0).reshape(g, n * _D).astype(jnp.float32)


@functools.partial(jax.jit, static_argnums=0)
def _run(g, y, lmat, l2mat, bwa, bw1, bw2, bias2d):
    b, f = y.shape
    return pl.pallas_call(
        _body,
        grid=(b // g,),
        in_specs=[
            pl.BlockSpec((g, f), lambda i: (i, 0)),
            pl.BlockSpec(lmat.shape, lambda i: (0, 0)),
            pl.BlockSpec(l2mat.shape, lambda i: (0, 0)),
            pl.BlockSpec((_C, _C), lambda i: (0, 0)),
            pl.BlockSpec((_C, _C), lambda i: (0, 0)),
            pl.BlockSpec((_C, _C), lambda i: (0, 0)),
            pl.BlockSpec((1, _C), lambda i: (0, 0)),
        ],
        out_specs=pl.BlockSpec((g, f), lambda i: (i, 0)),
        out_shape=jax.ShapeDtypeStruct((b, f), jnp.float32),
    )(y, lmat, l2mat, bwa, bw1, bw2, bias2d)


def kernel(t_local, y, L, W, b):
    del t_local
    d = W.shape[1]
    m = W.shape[0] // d
    eye = jnp.eye(_C // d, dtype=jnp.float32)
    w0, w1, w2 = (-0.1 * W[k::m, :] for k in range(m))
    bwa = jnp.kron(eye, w0 - w2).astype(jnp.bfloat16)
    bw1 = jnp.kron(eye, w1).astype(jnp.bfloat16)
    bw2 = jnp.kron(eye, w2).astype(jnp.bfloat16)
    bias2d = jnp.tile(-0.1 * b, _C // d).reshape(1, _C)
    return _run(_G, y, L.astype(jnp.bfloat16),
                (2.0 * L).astype(jnp.bfloat16), bwa, bw1, bw2, bias2d)
